# baseline (speedup 1.0000x reference)
import functools

import jax
import jax.numpy as jnp
from jax import lax
from jax.experimental import pallas as pl
from jax.experimental.pallas import tpu as pltpu, tpu_sc as plsc

N = 10000
E = 320000
D = 128
G = 128

NC = 2
NS = 16
NW = NC * NS
EPW = E // NW
K = 80
CH = EPW // K
RPT = 624
REM = N - RPT * NS
ZR = 39
NBUF = 3
NT = CH // NBUF
TAIL = CH - NBUF * NT

BN = 1000
NB = N // BN


def _sc_segment_partials(x, edges_r):
    mesh = plsc.VectorSubcoreMesh(core_axis_name="c", subcore_axis_name="s")

    @functools.partial(
        pl.kernel,
        out_type=jax.ShapeDtypeStruct((NC, N, D), jnp.float32),
        mesh=mesh,
        scratch_types=[
            pltpu.VMEM((NBUF, 2, K), jnp.int32),
            pltpu.VMEM((NBUF, 2, K), jnp.int32),
            [pltpu.VMEM((K,), jnp.int32) for _ in range(NBUF)],
            [pltpu.VMEM((K, D), jnp.float32) for _ in range(NBUF)],
            pltpu.VMEM((ZR, D), jnp.float32),
            pltpu.VMEM_SHARED((N, D), jnp.float32),
            [pltpu.SemaphoreType.DMA for _ in range(NBUF)],
            [pltpu.SemaphoreType.DMA for _ in range(NBUF)],
            pltpu.SemaphoreType.DMA,
        ],
    )
    def seg(x_hbm, e_hbm, out_hbm, ebuf, ebuf2, sibuf, rows,
            zbuf, acc, sg, ss, si):
        cid = lax.axis_index("c")
        sid = lax.axis_index("s")
        wid = sid * NC + cid

        zv = jnp.zeros((16,), jnp.float32)

        def zero_row(i, carry):
            for j in range(D // 16):
                zbuf[i, pl.ds(j * 16, 16)] = zv
            return carry

        lax.fori_loop(0, ZR, zero_row, 0)

        zdescs = [
            pltpu.async_copy(zbuf, acc.at[pl.ds(sid * RPT + t * ZR, ZR)], sg[0])
            for t in range(RPT // ZR)
        ]
        pltpu.sync_copy(e_hbm.at[wid, pl.ds(0, NBUF)], ebuf)
        for zd in zdescs:
            zd.wait()

        @pl.when(sid == NS - 1)
        def _():
            pltpu.sync_copy(zbuf.at[pl.ds(0, REM)],
                            acc.at[pl.ds(RPT * NS, REM)])

        plsc.subcore_barrier()


        def body(i, carry):
            pf = pltpu.async_copy(
                e_hbm.at[wid, pl.ds(NBUF * jnp.minimum(i + 1, NT - 1), NBUF)],
                ebuf2, si)

            gd = []
            for j in range(NBUF):
                @pl.when(i != 0)
                def _(j=j):
                    pltpu.make_async_copy(rows[j], acc.at[sibuf[j]],
                                          ss[j]).wait()
                for v in range(K // 16):
                    sibuf[j][pl.ds(16 * v, 16)] = ebuf[j, 1, pl.ds(16 * v, 16)]
                gd.append(
                    pltpu.async_copy(x_hbm.at[ebuf.at[j, 0]], rows[j], sg[j]))
            for j in range(NBUF):
                gd[j].wait()
                pltpu.async_copy(rows[j], acc.at[sibuf[j]], ss[j], add=True)
            pf.wait()
            for a in range(NBUF):
                for b in range(2):
                    for v in range(K // 16):
                        ebuf[a, b, pl.ds(16 * v, 16)] = \
                            ebuf2[a, b, pl.ds(16 * v, 16)]
            return carry

        lax.fori_loop(0, NT, body, 0)
        for j in range(NBUF):
            pltpu.make_async_copy(rows[j], acc.at[sibuf[j]], ss[j]).wait()

        for t in range(TAIL):
            pltpu.sync_copy(e_hbm.at[wid, pl.ds(CH - TAIL + t, 1)],
                            ebuf.at[pl.ds(0, 1)])
            pltpu.async_copy(x_hbm.at[ebuf.at[0, 0]], rows[0], sg[0]).wait()
            pltpu.sync_copy(rows[0], acc.at[ebuf.at[0, 1]], add=True)
        plsc.subcore_barrier()

        pltpu.sync_copy(acc.at[pl.ds(sid * RPT, RPT)],
                        out_hbm.at[cid, pl.ds(sid * RPT, RPT)])

        @pl.when(sid == NS - 1)
        def _():
            pltpu.sync_copy(acc.at[pl.ds(RPT * NS, REM)],
                            out_hbm.at[cid, pl.ds(RPT * NS, REM)])

    return seg(x, edges_r)


def _tc_layer(x, p, W, b):
    def body(x_ref, p0_ref, p1_ref, w_ref, b_ref, o_ref):
        s = x_ref[...] + p0_ref[0] + p1_ref[0]
        y = lax.dot(s, w_ref[...], preferred_element_type=jnp.float32)
        o_ref[...] = jnp.maximum(y + b_ref[...], 0.0)

    return pl.pallas_call(
        body,
        grid=(NB,),
        in_specs=[
            pl.BlockSpec((BN, D), lambda i: (i, 0)),
            pl.BlockSpec((1, BN, D), lambda i: (0, i, 0)),
            pl.BlockSpec((1, BN, D), lambda i: (1, i, 0)),
            pl.BlockSpec((D, D), lambda i: (0, 0)),
            pl.BlockSpec((1, D), lambda i: (0, 0)),
        ],
        out_specs=pl.BlockSpec((BN, D), lambda i: (i, 0)),
        out_shape=jax.ShapeDtypeStruct((N, D), jnp.float32),
    )(x, p, p, W, b)


def _tc_layer2_pool_proj(h, q, W2, b2, batch3, P1, pb1, P2, pb2):
    def body(h_ref, q0_ref, q1_ref, w_ref, b_ref, bat_ref,
             p1_ref, pb1_ref, p2_ref, pb2_ref, z_ref, acc, cnt):
        i = pl.program_id(0)
        s = h_ref[...] + q0_ref[0] + q1_ref[0]
        h2 = jnp.maximum(
            lax.dot(s, w_ref[...], preferred_element_type=jnp.float32)
            + b_ref[...], 0.0)
        bb = bat_ref[0, 0, :]
        gids = lax.broadcasted_iota(jnp.int32, (G, BN), 0)
        onehot_t = (gids == bb[None, :]).astype(jnp.float32)

        @pl.when(i == 0)
        def _():
            acc[...] = jnp.zeros_like(acc)
            cnt[...] = jnp.zeros_like(cnt)

        acc[...] += lax.dot(onehot_t, h2, preferred_element_type=jnp.float32)
        cnt[...] += lax.dot(onehot_t, jnp.ones((BN, D), jnp.float32),
                            preferred_element_type=jnp.float32)

        @pl.when(i == NB - 1)
        def _():
            pooled = acc[...] / jnp.maximum(cnt[...], 1.0)
            t = jnp.maximum(
                lax.dot(pooled, p1_ref[...], preferred_element_type=jnp.float32)
                + pb1_ref[...], 0.0)
            z_ref[...] = (lax.dot(t, p2_ref[...],
                                  preferred_element_type=jnp.float32)
                          + pb2_ref[...])

    return pl.pallas_call(
        body,
        grid=(NB,),
        in_specs=[
            pl.BlockSpec((BN, D), lambda i: (i, 0)),
            pl.BlockSpec((1, BN, D), lambda i: (0, i, 0)),
            pl.BlockSpec((1, BN, D), lambda i: (1, i, 0)),
            pl.BlockSpec((D, D), lambda i: (0, 0)),
            pl.BlockSpec((1, D), lambda i: (0, 0)),
            pl.BlockSpec((1, 1, BN), lambda i: (i, 0, 0)),
            pl.BlockSpec((D, D), lambda i: (0, 0)),
            pl.BlockSpec((1, D), lambda i: (0, 0)),
            pl.BlockSpec((D, D), lambda i: (0, 0)),
            pl.BlockSpec((1, D), lambda i: (0, 0)),
        ],
        out_specs=pl.BlockSpec((G, D), lambda i: (0, 0)),
        out_shape=jax.ShapeDtypeStruct((G, D), jnp.float32),
        scratch_shapes=[pltpu.VMEM((G, D), jnp.float32),
                        pltpu.VMEM((G, D), jnp.float32)],
    )(h, q, q, W2, b2, batch3, P1, pb1, P2, pb2)


def kernel(x, edge_index, batch, W1, b1, W2, b2, P1, pb1, P2, pb2):
    edges_r = jnp.stack(
        [edge_index[0].reshape(NW, CH, K), edge_index[1].reshape(NW, CH, K)],
        axis=2)
    batch3 = batch.reshape(NB, 1, BN)
    b1r = b1.reshape(1, D)
    b2r = b2.reshape(1, D)
    pb1r = pb1.reshape(1, D)
    pb2r = pb2.reshape(1, D)

    p = _sc_segment_partials(x, edges_r)
    h = _tc_layer(x, p, W1, b1r)
    q = _sc_segment_partials(h, edges_r)
    z = _tc_layer2_pool_proj(h, q, W2, b2r, batch3, P1, pb1r, P2, pb2r)
    return z


# speedup vs baseline: 1.0853x; 1.0853x over previous
import functools

import jax
import jax.numpy as jnp
from jax import lax
from jax.experimental import pallas as pl
from jax.experimental.pallas import tpu as pltpu, tpu_sc as plsc

N = 10000
E = 320000
D = 128
G = 128

NC = 2
NS = 16
NW = NC * NS
EPW = E // NW
K = 80
CH = EPW // K
RPT = 624
REM = N - RPT * NS
ZR = 39
NBUF = 3
NT = CH // NBUF
TAIL = CH - NBUF * NT

BN = 1000
NB = N // BN


def _sc_segment_partials(x, edge_flat):
    mesh = plsc.VectorSubcoreMesh(core_axis_name="c", subcore_axis_name="s")
    TK = NBUF * K

    @functools.partial(
        pl.kernel,
        out_type=jax.ShapeDtypeStruct((NC, N, D), jnp.float32),
        mesh=mesh,
        scratch_types=[
            pltpu.VMEM((TK,), jnp.int32),
            pltpu.VMEM((TK,), jnp.int32),
            pltpu.VMEM((TK,), jnp.int32),
            pltpu.VMEM((TK,), jnp.int32),
            [pltpu.VMEM((K,), jnp.int32) for _ in range(NBUF)],
            [pltpu.VMEM((K, D), jnp.float32) for _ in range(NBUF)],
            pltpu.VMEM((ZR, D), jnp.float32),
            pltpu.VMEM_SHARED((N, D), jnp.float32),
            [pltpu.SemaphoreType.DMA for _ in range(NBUF)],
            [pltpu.SemaphoreType.DMA for _ in range(NBUF)],
            pltpu.SemaphoreType.DMA,
            pltpu.SemaphoreType.DMA,
        ],
    )
    def seg(x_hbm, e_hbm, out_hbm, ebufS, ebufD, ebufS2, ebufD2, sibuf, rows,
            zbuf, acc, sg, ss, siS, siD):
        cid = lax.axis_index("c")
        sid = lax.axis_index("s")
        wid = sid * NC + cid

        zv = jnp.zeros((16,), jnp.float32)

        def zero_row(i, carry):
            for j in range(D // 16):
                zbuf[i, pl.ds(j * 16, 16)] = zv
            return carry

        lax.fori_loop(0, ZR, zero_row, 0)

        base = wid * EPW

        zdescs = [
            pltpu.async_copy(zbuf, acc.at[pl.ds(sid * RPT + t * ZR, ZR)], sg[0])
            for t in range(RPT // ZR)
        ]
        pltpu.sync_copy(e_hbm.at[pl.ds(base, TK)], ebufS)
        pltpu.sync_copy(e_hbm.at[pl.ds(E + base, TK)], ebufD)
        for zd in zdescs:
            zd.wait()

        @pl.when(sid == NS - 1)
        def _():
            pltpu.sync_copy(zbuf.at[pl.ds(0, REM)],
                            acc.at[pl.ds(RPT * NS, REM)])

        plsc.subcore_barrier()


        def body(i, carry):
            noff = base + TK * jnp.minimum(i + 1, NT - 1)
            pfS = pltpu.async_copy(e_hbm.at[pl.ds(noff, TK)], ebufS2, siS)
            pfD = pltpu.async_copy(e_hbm.at[pl.ds(E + noff, TK)], ebufD2, siD)

            gd = []
            for j in range(NBUF):
                @pl.when(i != 0)
                def _(j=j):
                    pltpu.make_async_copy(rows[j], acc.at[sibuf[j]],
                                          ss[j]).wait()
                for v in range(K // 16):
                    sibuf[j][pl.ds(16 * v, 16)] = \
                        ebufD[pl.ds(j * K + 16 * v, 16)]
                gd.append(pltpu.async_copy(
                    x_hbm.at[ebufS.at[pl.ds(j * K, K)]], rows[j], sg[j]))
            for j in range(NBUF):
                gd[j].wait()
                pltpu.async_copy(rows[j], acc.at[sibuf[j]], ss[j], add=True)
            pfS.wait()
            pfD.wait()
            for v in range(TK // 16):
                ebufS[pl.ds(16 * v, 16)] = ebufS2[pl.ds(16 * v, 16)]
                ebufD[pl.ds(16 * v, 16)] = ebufD2[pl.ds(16 * v, 16)]
            return carry

        lax.fori_loop(0, NT, body, 0)
        for j in range(NBUF):
            pltpu.make_async_copy(rows[j], acc.at[sibuf[j]], ss[j]).wait()

        for t in range(TAIL):
            coff = base + (CH - TAIL + t) * K
            pltpu.sync_copy(e_hbm.at[pl.ds(coff, K)],
                            ebufS.at[pl.ds(0, K)])
            pltpu.sync_copy(e_hbm.at[pl.ds(E + coff, K)], sibuf[0])
            pltpu.async_copy(x_hbm.at[ebufS.at[pl.ds(0, K)]],
                             rows[0], sg[0]).wait()
            pltpu.sync_copy(rows[0], acc.at[sibuf[0]], add=True)
        plsc.subcore_barrier()

        pltpu.sync_copy(acc.at[pl.ds(sid * RPT, RPT)],
                        out_hbm.at[cid, pl.ds(sid * RPT, RPT)])

        @pl.when(sid == NS - 1)
        def _():
            pltpu.sync_copy(acc.at[pl.ds(RPT * NS, REM)],
                            out_hbm.at[cid, pl.ds(RPT * NS, REM)])

    return seg(x, edge_flat)


def _tc_layer(x, p, W, b):
    def body(x_ref, p0_ref, p1_ref, w_ref, b_ref, o_ref):
        s = x_ref[...] + p0_ref[0] + p1_ref[0]
        y = lax.dot(s, w_ref[...], preferred_element_type=jnp.float32)
        o_ref[...] = jnp.maximum(y + b_ref[...], 0.0)

    return pl.pallas_call(
        body,
        grid=(NB,),
        in_specs=[
            pl.BlockSpec((BN, D), lambda i: (i, 0)),
            pl.BlockSpec((1, BN, D), lambda i: (0, i, 0)),
            pl.BlockSpec((1, BN, D), lambda i: (1, i, 0)),
            pl.BlockSpec((D, D), lambda i: (0, 0)),
            pl.BlockSpec((1, D), lambda i: (0, 0)),
        ],
        out_specs=pl.BlockSpec((BN, D), lambda i: (i, 0)),
        out_shape=jax.ShapeDtypeStruct((N, D), jnp.float32),
    )(x, p, p, W, b)


def _tc_layer2_pool_proj(h, q, W2, b2, batch3, P1, pb1, P2, pb2):
    def body(h_ref, q0_ref, q1_ref, w_ref, b_ref, bat_ref,
             p1_ref, pb1_ref, p2_ref, pb2_ref, z_ref, acc, cnt):
        i = pl.program_id(0)
        s = h_ref[...] + q0_ref[0] + q1_ref[0]
        h2 = jnp.maximum(
            lax.dot(s, w_ref[...], preferred_element_type=jnp.float32)
            + b_ref[...], 0.0)
        bb = bat_ref[0, 0, :]
        gids = lax.broadcasted_iota(jnp.int32, (G, BN), 0)
        onehot_t = (gids == bb[None, :]).astype(jnp.float32)

        @pl.when(i == 0)
        def _():
            acc[...] = jnp.zeros_like(acc)
            cnt[...] = jnp.zeros_like(cnt)

        acc[...] += lax.dot(onehot_t, h2, preferred_element_type=jnp.float32)
        cnt[...] += lax.dot(onehot_t, jnp.ones((BN, D), jnp.float32),
                            preferred_element_type=jnp.float32)

        @pl.when(i == NB - 1)
        def _():
            pooled = acc[...] / jnp.maximum(cnt[...], 1.0)
            t = jnp.maximum(
                lax.dot(pooled, p1_ref[...], preferred_element_type=jnp.float32)
                + pb1_ref[...], 0.0)
            z_ref[...] = (lax.dot(t, p2_ref[...],
                                  preferred_element_type=jnp.float32)
                          + pb2_ref[...])

    return pl.pallas_call(
        body,
        grid=(NB,),
        in_specs=[
            pl.BlockSpec((BN, D), lambda i: (i, 0)),
            pl.BlockSpec((1, BN, D), lambda i: (0, i, 0)),
            pl.BlockSpec((1, BN, D), lambda i: (1, i, 0)),
            pl.BlockSpec((D, D), lambda i: (0, 0)),
            pl.BlockSpec((1, D), lambda i: (0, 0)),
            pl.BlockSpec((1, 1, BN), lambda i: (i, 0, 0)),
            pl.BlockSpec((D, D), lambda i: (0, 0)),
            pl.BlockSpec((1, D), lambda i: (0, 0)),
            pl.BlockSpec((D, D), lambda i: (0, 0)),
            pl.BlockSpec((1, D), lambda i: (0, 0)),
        ],
        out_specs=pl.BlockSpec((G, D), lambda i: (0, 0)),
        out_shape=jax.ShapeDtypeStruct((G, D), jnp.float32),
        scratch_shapes=[pltpu.VMEM((G, D), jnp.float32),
                        pltpu.VMEM((G, D), jnp.float32)],
    )(h, q, q, W2, b2, batch3, P1, pb1, P2, pb2)


def kernel(x, edge_index, batch, W1, b1, W2, b2, P1, pb1, P2, pb2):
    batch3 = batch.reshape(NB, 1, BN)
    b1r = b1.reshape(1, D)
    b2r = b2.reshape(1, D)
    pb1r = pb1.reshape(1, D)
    pb2r = pb2.reshape(1, D)

    edge_flat = edge_index.reshape(2 * E)
    p = _sc_segment_partials(x, edge_flat)
    h = _tc_layer(x, p, W1, b1r)
    q = _sc_segment_partials(h, edge_flat)
    z = _tc_layer2_pool_proj(h, q, W2, b2r, batch3, P1, pb1r, P2, pb2r)
    return z
